# baseline (speedup 1.0000x reference)
import functools

import jax
import jax.numpy as jnp
from jax import lax
from jax.experimental import pallas as pl
from jax.experimental.pallas import tpu as pltpu
from jax.experimental.pallas import tpu_sc as plsc

NBITS = 1 << 24
MASK = NBITS - 1
PRIME_I32 = 2654435761 - (1 << 32)

N_ADD = 1_000_000
N_ADD_PAD = 1 << 20
N_Q = 1 << 22

NUM_CORES = 2
NUM_SUBCORES = 16
NW = NUM_CORES * NUM_SUBCORES
LB = 128
CR = 8

def _hash_rows(vals_ref, idx_ref):
  for j in range(CR):
    for l in range(LB // 16):
      v = vals_ref[j, pl.ds(l * 16, 16)]
      idx_ref[j, pl.ds(l * 16, 16)] = (v * jnp.int32(PRIME_I32)) & jnp.int32(MASK)


def _scatter_body(vals_hbm, s_hbm, vals_v, idx_v, ones_v, sem):
  wid = lax.axis_index("s") * NUM_CORES + lax.axis_index("c")
  rows_per_tile = N_ADD_PAD // LB // NW
  row0 = wid * rows_per_tile
  for l in range(LB // 16):
    ones_v[pl.ds(l * 16, 16)] = jnp.full((16,), 1, jnp.int32)

  @pl.loop(0, rows_per_tile // CR)
  def _chunk(ci):
    r = row0 + ci * CR
    pltpu.sync_copy(vals_hbm.at[pl.ds(r, CR)], vals_v)
    _hash_rows(vals_v, idx_v)
    copies = [
        pltpu.async_copy(ones_v, s_hbm.at[idx_v.at[j]], sem) for j in range(CR)
    ]
    for cp in copies:
      cp.wait()


NWORDS = NBITS // 32


def _gather_body(qvals_hbm, wp_hbm, out_hbm, qv, qidx, qbit, res, wsh, sem):
  cid = lax.axis_index("c")
  sid = lax.axis_index("s")
  wid = sid * NUM_CORES + cid
  rows_per_tile = N_Q // LB // NW
  row0 = wid * rows_per_tile

  stage = NWORDS // NUM_SUBCORES
  pltpu.sync_copy(wp_hbm.at[pl.ds(sid * stage, stage)],
                  wsh.at[pl.ds(sid * stage, stage)])
  plsc.subcore_barrier()

  @pl.loop(0, rows_per_tile // CR)
  def _chunk(ci):
    r = row0 + ci * CR
    pltpu.sync_copy(qvals_hbm.at[pl.ds(r, CR)], qv)
    for j in range(CR):
      for l in range(LB // 16):
        v = qv[j, pl.ds(l * 16, 16)]
        h = (v * jnp.int32(PRIME_I32)) & jnp.int32(MASK)
        qidx[j, pl.ds(l * 16, 16)] = h >> 5
        qbit[j, pl.ds(l * 16, 16)] = h & 31
    copies = [
        pltpu.async_copy(wsh.at[qidx.at[j]], res.at[j], sem)
        for j in range(CR)
    ]
    for cp in copies:
      cp.wait()
    for j in range(CR):
      for l in range(LB // 16):
        w = res[j, pl.ds(l * 16, 16)]
        b = qbit[j, pl.ds(l * 16, 16)]
        res[j, pl.ds(l * 16, 16)] = (w >> b) & 1
    pltpu.sync_copy(res, out_hbm.at[pl.ds(r, CR)])


@functools.cache
def _sc_kernels():
  mesh = plsc.VectorSubcoreMesh(
      core_axis_name="c", subcore_axis_name="s",
      num_cores=NUM_CORES, num_subcores=NUM_SUBCORES)
  scatter = pl.kernel(
      _scatter_body,
      mesh=mesh,
      scratch_types=[
          pltpu.VMEM((CR, LB), jnp.int32),
          pltpu.VMEM((CR, LB), jnp.int32),
          pltpu.VMEM((LB,), jnp.int32),
          pltpu.SemaphoreType.DMA,
      ],
  )
  gather = pl.kernel(
      _gather_body,
      out_type=jax.ShapeDtypeStruct((N_Q // LB, LB), jnp.int32),
      mesh=mesh,
      scratch_types=[
          pltpu.VMEM((CR, LB), jnp.int32),
          pltpu.VMEM((CR, LB), jnp.int32),
          pltpu.VMEM((CR, LB), jnp.int32),
          pltpu.VMEM((CR, LB), jnp.int32),
          pltpu.VMEM_SHARED((NWORDS,), jnp.int32),
          pltpu.SemaphoreType.DMA,
      ],
  )
  return scatter, gather


R2D = 16384
C2D = 1024
RB = 512
NBLK = R2D // RB


def _window_body(x_ref, prev_ref, next_ref, o_ref):
  X = x_ref[...]
  Xe = jnp.concatenate([prev_ref[0], X, next_ref[0]], axis=0)
  tail = jnp.concatenate([Xe[:1, C2D - 8:], Xe[:-1, C2D - 8:]], axis=0)
  E1 = jnp.concatenate([tail, Xe], axis=1)
  A = E1[:, 8:]
  for e in range(1, 7):
    A = A | E1[:, 8 - e:8 - e + C2D]
  head = jnp.concatenate([A[1:, :8], A[-1:, :8]], axis=0)
  E2 = jnp.concatenate([A, head], axis=1)
  W = E2[:, :C2D]
  for d in range(1, 7):
    W = W & E2[:, d:d + C2D]
  Wb = W[1:RB + 1]
  W3 = Wb.reshape(RB, C2D // 32, 32)
  shifts = jax.lax.broadcasted_iota(jnp.int32, (RB, C2D // 32, 32), 2)
  o_ref[...] = jnp.sum(W3 << shifts, axis=2, dtype=jnp.int32)


_window = pl.pallas_call(
    _window_body,
    grid=(NBLK,),
    in_specs=[
        pl.BlockSpec((RB, C2D), lambda i: (i, 0)),
        pl.BlockSpec((1, 1, C2D), lambda i: (i, 0, 0)),
        pl.BlockSpec((1, 1, C2D), lambda i: (i, 0, 0)),
    ],
    out_specs=pl.BlockSpec((RB, C2D // 32), lambda i: (i, 0)),
    out_shape=jax.ShapeDtypeStruct((R2D, C2D // 32), jnp.int32),
)


def kernel(values_add, values_query):
  pad = jnp.broadcast_to(values_add[0], (N_ADD_PAD - N_ADD,))
  vals2d = jnp.concatenate([values_add, pad]).reshape(N_ADD_PAD // LB, LB)
  q2d = values_query.reshape(N_Q // LB, LB)

  scatter_markers, gather_queries = _sc_kernels()
  s_ref = jax.new_ref(jnp.zeros((NBITS,), jnp.int32))
  scatter_markers(vals2d, s_ref)
  S2 = s_ref[...].reshape(R2D, C2D)

  prev_rows = jnp.roll(S2[RB - 1::RB], 1, axis=0).reshape(NBLK, 1, C2D)
  next_rows = jnp.roll(S2[0::RB], -1, axis=0).reshape(NBLK, 1, C2D)
  W2 = _window(S2, prev_rows, next_rows)

  out2d = gather_queries(q2d, W2.reshape(-1))
  return out2d.reshape(-1) != 0


# speedup vs baseline: 14.9862x; 14.9862x over previous
import functools

import jax
import jax.numpy as jnp
from jax import lax
from jax.experimental import pallas as pl
from jax.experimental.pallas import tpu as pltpu
from jax.experimental.pallas import tpu_sc as plsc

NBITS = 1 << 24
MASK = NBITS - 1
PRIME_I32 = 2654435761 - (1 << 32)

N_ADD = 1_000_000
N_ADD_PAD = 1 << 20
N_Q = 1 << 22

NUM_CORES = 2
NUM_SUBCORES = 16
NW = NUM_CORES * NUM_SUBCORES
LB = 128
CR = 8

NWORDS = NBITS // 32

GROUPS = 8
REPL = 4
GBITS = NBITS // GROUPS
GWORDS = GBITS // 32
SENT_LO = GWORDS * 32
VCH = 8192


def _scatter_body(vals_hbm, out_hbm, vbuf, stage, bset):
  wid = lax.axis_index("s") * NUM_CORES + lax.axis_index("c")
  g = wid >> 2
  r = wid & 3
  lo0 = g * GBITS
  iota = jnp.arange(16, dtype=jnp.int32)
  DUMP = jnp.int32(GWORDS)
  SCAP = VCH // 16
  SDUMP = jnp.int32(16 * SCAP)

  @pl.loop(0, (GWORDS + 16) // 16)
  def _zero(z):
    bset[pl.ds(z * 16, 16)] = jnp.zeros((16,), jnp.int32)

  vals_per_repl = N_ADD_PAD // REPL
  base0 = r * vals_per_repl

  @pl.loop(0, vals_per_repl // VCH)
  def _chunk(ci):
    pltpu.sync_copy(vals_hbm.at[pl.ds(base0 + ci * VCH, VCH)], vbuf)

    @pl.loop(0, VCH // 16, init_carry=jnp.zeros((16,), jnp.int32))
    def _scan(vi, cnt):
      v = vbuf[pl.ds(vi * 16, 16)]
      h = (v * jnp.int32(PRIME_I32)) & jnp.int32(MASK)
      lo = h - lo0
      inr = (lo >= 0) & (lo < GBITS)
      tgt = jnp.where(inr, cnt * 16 + iota, SDUMP + iota)
      plsc.store_scatter(stage, [tgt], lo)
      return cnt + jnp.where(inr, 1, 0)

    cnt = _scan
    nmax = cnt[0]
    for i in range(1, 16):
      nmax = jnp.maximum(nmax, cnt[i])

    @pl.loop(0, nmax)
    def _drain(e):
      raw = stage[pl.ds(e * 16, 16)]
      valid = e < cnt
      w = jnp.where(valid, raw >> 5, DUMP)
      m = jnp.where(valid, jnp.left_shift(jnp.int32(1), raw & 31), 0)
      w, m = lax.sort([w, m], num_keys=1)
      for s in (1, 2, 4, 8):
        pd = jnp.minimum(iota + s, 15)
        w_d = jnp.take_along_axis(w, pd, axis=0, mode="promise_in_bounds")
        m_d = jnp.take_along_axis(m, pd, axis=0, mode="promise_in_bounds")
        m = m | jnp.where(w_d == w, m_d, 0)
      w_u = jnp.take_along_axis(
          w, jnp.maximum(iota - 1, 0), axis=0, mode="promise_in_bounds")
      w = jnp.where((iota == 0) | (w != w_u), w, DUMP)
      cur = plsc.load_gather(bset, [w])
      plsc.store_scatter(bset, [w], cur | m)

  pltpu.sync_copy(bset.at[pl.ds(0, GWORDS)], out_hbm.at[wid])


def _gather_body(qvals_hbm, wp_hbm, out_hbm, qv, qidx, qbit, res, wsh, sem):
  cid = lax.axis_index("c")
  sid = lax.axis_index("s")
  wid = sid * NUM_CORES + cid
  rows_per_tile = N_Q // LB // NW
  row0 = wid * rows_per_tile

  stage = NWORDS // NUM_SUBCORES
  pltpu.sync_copy(wp_hbm.at[pl.ds(sid * stage, stage)],
                  wsh.at[pl.ds(sid * stage, stage)])
  plsc.subcore_barrier()

  @pl.loop(0, rows_per_tile // CR)
  def _chunk(ci):
    r = row0 + ci * CR
    pltpu.sync_copy(qvals_hbm.at[pl.ds(r, CR)], qv)
    for j in range(CR):
      for l in range(LB // 16):
        v = qv[j, pl.ds(l * 16, 16)]
        h = (v * jnp.int32(PRIME_I32)) & jnp.int32(MASK)
        qidx[j, pl.ds(l * 16, 16)] = h >> 5
        qbit[j, pl.ds(l * 16, 16)] = h & 31
    copies = [
        pltpu.async_copy(wsh.at[qidx.at[j]], res.at[j], sem)
        for j in range(CR)
    ]
    for cp in copies:
      cp.wait()
    for j in range(CR):
      for l in range(LB // 16):
        w = res[j, pl.ds(l * 16, 16)]
        b = qbit[j, pl.ds(l * 16, 16)]
        res[j, pl.ds(l * 16, 16)] = (w >> b) & 1
    pltpu.sync_copy(res, out_hbm.at[pl.ds(r, CR)])


@functools.cache
def _sc_kernels():
  mesh = plsc.VectorSubcoreMesh(
      core_axis_name="c", subcore_axis_name="s",
      num_cores=NUM_CORES, num_subcores=NUM_SUBCORES)
  scatter = pl.kernel(
      _scatter_body,
      out_type=jax.ShapeDtypeStruct((NW, GWORDS), jnp.int32),
      mesh=mesh,
      compiler_params=pltpu.CompilerParams(needs_layout_passes=False),
      scratch_types=[
          pltpu.VMEM((VCH,), jnp.int32),
          pltpu.VMEM((VCH + 16,), jnp.int32),
          pltpu.VMEM((GWORDS + 16,), jnp.int32),
      ],
  )
  gather = pl.kernel(
      _gather_body,
      out_type=jax.ShapeDtypeStruct((N_Q // LB, LB), jnp.int32),
      mesh=mesh,
      scratch_types=[
          pltpu.VMEM((CR, LB), jnp.int32),
          pltpu.VMEM((CR, LB), jnp.int32),
          pltpu.VMEM((CR, LB), jnp.int32),
          pltpu.VMEM((CR, LB), jnp.int32),
          pltpu.VMEM_SHARED((NWORDS,), jnp.int32),
          pltpu.SemaphoreType.DMA,
      ],
  )
  return scatter, gather


R2D = 512
C2D = 1024


def _window_body(x0, x1, x2, x3, o_ref):
  S = x0[...] | x1[...] | x2[...] | x3[...]
  cl = S[:, C2D - 1:]
  Sd = jnp.concatenate([cl[R2D - 1:], cl[:R2D - 1]], axis=0)
  P = jnp.concatenate([Sd, S[:, :C2D - 1]], axis=1)
  A = S
  for e in range(1, 7):
    A = A | (S << e) | lax.shift_right_logical(P, 32 - e)
  c0 = A[:, :1]
  Au = jnp.concatenate([c0[1:], c0[:1]], axis=0)
  Nx = jnp.concatenate([A[:, 1:], Au], axis=1)
  W = A
  for d in range(1, 7):
    W = W & (lax.shift_right_logical(A, d) | (Nx << (32 - d)))
  o_ref[...] = W


_window = pl.pallas_call(
    _window_body,
    out_shape=jax.ShapeDtypeStruct((R2D, C2D), jnp.int32),
)


def kernel(values_add, values_query):
  pad = jnp.broadcast_to(values_add[0], (N_ADD_PAD - N_ADD,))
  vals1d = jnp.concatenate([values_add, pad])
  q2d = values_query.reshape(N_Q // LB, LB)

  scatter_markers, gather_queries = _sc_kernels()
  reps32 = scatter_markers(vals1d)
  reps = reps32.reshape(GROUPS, REPL, GWORDS)
  rep = [reps[:, i, :].reshape(R2D, C2D) for i in range(REPL)]
  Wp = _window(rep[0], rep[1], rep[2], rep[3])

  out2d = gather_queries(q2d, Wp.reshape(-1))
  return out2d.reshape(-1) != 0


# speedup vs baseline: 15.6886x; 1.0469x over previous
import functools

import jax
import jax.numpy as jnp
from jax import lax
from jax.experimental import pallas as pl
from jax.experimental.pallas import tpu as pltpu
from jax.experimental.pallas import tpu_sc as plsc

NBITS = 1 << 24
MASK = NBITS - 1
PRIME_I32 = 2654435761 - (1 << 32)

N_ADD = 1_000_000
N_ADD_PAD = 1 << 20
N_Q = 1 << 22

NUM_CORES = 2
NUM_SUBCORES = 16
NW = NUM_CORES * NUM_SUBCORES
LB = 128
CR = 8

NWORDS = NBITS // 32

GROUPS = 8
REPL = 4
GBITS = NBITS // GROUPS
GWORDS = GBITS // 32
SENT_LO = GWORDS * 32
VCH = 8192


def _scatter_body(vals_hbm, out_hbm, vbuf0, vbuf1, stage, bset, sem0, sem1):
  wid = lax.axis_index("s") * NUM_CORES + lax.axis_index("c")
  g = wid >> 2
  r = wid & 3
  lo0 = g * GBITS
  iota = jnp.arange(16, dtype=jnp.int32)
  DUMP = jnp.int32(GWORDS)
  SCAP = VCH // 16
  SDUMP = jnp.int32(16 * SCAP)

  @pl.loop(0, (GWORDS + 16) // 16)
  def _zero(z):
    bset[pl.ds(z * 16, 16)] = jnp.zeros((16,), jnp.int32)

  vals_per_repl = N_ADD_PAD // REPL
  base0 = r * vals_per_repl
  nchunks = vals_per_repl // VCH

  def _process(vbuf_s, sem_s, ci):
    pltpu.make_async_copy(vals_hbm.at[pl.ds(0, VCH)], vbuf_s, sem_s).wait()

    @pl.when(ci + 2 < nchunks)
    def _prefetch():
      pltpu.async_copy(
          vals_hbm.at[pl.ds(base0 + (ci + 2) * VCH, VCH)], vbuf_s, sem_s)

    @pl.loop(0, VCH // 16, init_carry=jnp.zeros((16,), jnp.int32), unroll=4)
    def _scan(vi, cnt):
      v = vbuf_s[pl.ds(vi * 16, 16)]
      h = (v * jnp.int32(PRIME_I32)) & jnp.int32(MASK)
      lo = h - lo0
      inr = (lo >= 0) & (lo < GBITS)
      tgt = jnp.where(inr, cnt * 16 + iota, SDUMP + iota)
      plsc.store_scatter(stage, [tgt], lo)
      return cnt + jnp.where(inr, 1, 0)

    cnt = _scan
    nmax = cnt[0]
    for i in range(1, 16):
      nmax = jnp.maximum(nmax, cnt[i])

    @pl.loop(0, nmax)
    def _drain(e):
      raw = stage[pl.ds(e * 16, 16)]
      valid = e < cnt
      w = jnp.where(valid, raw >> 5, DUMP)
      m = jnp.where(valid, jnp.left_shift(jnp.int32(1), raw & 31), 0)
      w, m = lax.sort([w, m], num_keys=1)
      for s in (1, 2, 4, 8):
        pd = jnp.minimum(iota + s, 15)
        w_d = jnp.take_along_axis(w, pd, axis=0, mode="promise_in_bounds")
        m_d = jnp.take_along_axis(m, pd, axis=0, mode="promise_in_bounds")
        m = m | jnp.where(w_d == w, m_d, 0)
      w_u = jnp.take_along_axis(
          w, jnp.maximum(iota - 1, 0), axis=0, mode="promise_in_bounds")
      w = jnp.where((iota == 0) | (w != w_u), w, DUMP)
      cur = plsc.load_gather(bset, [w])
      plsc.store_scatter(bset, [w], cur | m)

  vb0, vb1 = vbuf0, vbuf1
  pltpu.async_copy(vals_hbm.at[pl.ds(base0, VCH)], vb0, sem0)
  pltpu.async_copy(vals_hbm.at[pl.ds(base0 + VCH, VCH)], vb1, sem1)

  @pl.loop(0, nchunks // 2)
  def _chunk(pi):
    _process(vb0, sem0, pi * 2)
    _process(vb1, sem1, pi * 2 + 1)

  pltpu.sync_copy(bset.at[pl.ds(0, GWORDS)], out_hbm.at[wid])


def _gather_body(qvals_hbm, wp_hbm, out_hbm, qv, qidx, qbit, res, wsh, sem):
  cid = lax.axis_index("c")
  sid = lax.axis_index("s")
  wid = sid * NUM_CORES + cid
  rows_per_tile = N_Q // LB // NW
  row0 = wid * rows_per_tile

  stage = NWORDS // NUM_SUBCORES
  pltpu.sync_copy(wp_hbm.at[pl.ds(sid * stage, stage)],
                  wsh.at[pl.ds(sid * stage, stage)])
  plsc.subcore_barrier()

  @pl.loop(0, rows_per_tile // CR)
  def _chunk(ci):
    r = row0 + ci * CR
    pltpu.sync_copy(qvals_hbm.at[pl.ds(r, CR)], qv)
    for j in range(CR):
      for l in range(LB // 16):
        v = qv[j, pl.ds(l * 16, 16)]
        h = (v * jnp.int32(PRIME_I32)) & jnp.int32(MASK)
        qidx[j, pl.ds(l * 16, 16)] = h >> 5
        qbit[j, pl.ds(l * 16, 16)] = h & 31
    copies = [
        pltpu.async_copy(wsh.at[qidx.at[j]], res.at[j], sem)
        for j in range(CR)
    ]
    for cp in copies:
      cp.wait()
    for j in range(CR):
      for l in range(LB // 16):
        w = res[j, pl.ds(l * 16, 16)]
        b = qbit[j, pl.ds(l * 16, 16)]
        res[j, pl.ds(l * 16, 16)] = (w >> b) & 1
    pltpu.sync_copy(res, out_hbm.at[pl.ds(r, CR)])


@functools.cache
def _sc_kernels():
  mesh = plsc.VectorSubcoreMesh(
      core_axis_name="c", subcore_axis_name="s",
      num_cores=NUM_CORES, num_subcores=NUM_SUBCORES)
  scatter = pl.kernel(
      _scatter_body,
      out_type=jax.ShapeDtypeStruct((NW, GWORDS), jnp.int32),
      mesh=mesh,
      compiler_params=pltpu.CompilerParams(needs_layout_passes=False),
      scratch_types=[
          pltpu.VMEM((VCH,), jnp.int32),
          pltpu.VMEM((VCH,), jnp.int32),
          pltpu.VMEM((VCH + 16,), jnp.int32),
          pltpu.VMEM((GWORDS + 16,), jnp.int32),
          pltpu.SemaphoreType.DMA,
          pltpu.SemaphoreType.DMA,
      ],
  )
  gather = pl.kernel(
      _gather_body,
      out_type=jax.ShapeDtypeStruct((N_Q // LB, LB), jnp.int32),
      mesh=mesh,
      scratch_types=[
          pltpu.VMEM((CR, LB), jnp.int32),
          pltpu.VMEM((CR, LB), jnp.int32),
          pltpu.VMEM((CR, LB), jnp.int32),
          pltpu.VMEM((CR, LB), jnp.int32),
          pltpu.VMEM_SHARED((NWORDS,), jnp.int32),
          pltpu.SemaphoreType.DMA,
      ],
  )
  return scatter, gather


R2D = 512
C2D = 1024


def _window_body(x0, x1, x2, x3, o_ref):
  S = x0[...] | x1[...] | x2[...] | x3[...]
  cl = S[:, C2D - 1:]
  Sd = jnp.concatenate([cl[R2D - 1:], cl[:R2D - 1]], axis=0)
  P = jnp.concatenate([Sd, S[:, :C2D - 1]], axis=1)
  A = S
  for e in range(1, 7):
    A = A | (S << e) | lax.shift_right_logical(P, 32 - e)
  c0 = A[:, :1]
  Au = jnp.concatenate([c0[1:], c0[:1]], axis=0)
  Nx = jnp.concatenate([A[:, 1:], Au], axis=1)
  W = A
  for d in range(1, 7):
    W = W & (lax.shift_right_logical(A, d) | (Nx << (32 - d)))
  o_ref[...] = W


_window = pl.pallas_call(
    _window_body,
    out_shape=jax.ShapeDtypeStruct((R2D, C2D), jnp.int32),
)


def kernel(values_add, values_query):
  pad = jnp.broadcast_to(values_add[0], (N_ADD_PAD - N_ADD,))
  vals1d = jnp.concatenate([values_add, pad])
  q2d = values_query.reshape(N_Q // LB, LB)

  scatter_markers, gather_queries = _sc_kernels()
  reps32 = scatter_markers(vals1d)
  reps = reps32.reshape(GROUPS, REPL, GWORDS)
  rep = [reps[:, i, :].reshape(R2D, C2D) for i in range(REPL)]
  Wp = _window(rep[0], rep[1], rep[2], rep[3])

  out2d = gather_queries(q2d, Wp.reshape(-1))
  return out2d.reshape(-1) != 0


# speedup vs baseline: 16.2134x; 1.0335x over previous
import functools

import jax
import jax.numpy as jnp
from jax import lax
from jax.experimental import pallas as pl
from jax.experimental.pallas import tpu as pltpu
from jax.experimental.pallas import tpu_sc as plsc

NBITS = 1 << 24
MASK = NBITS - 1
PRIME_I32 = 2654435761 - (1 << 32)

N_ADD = 1_000_000
N_ADD_PAD = 1 << 20
N_Q = 1 << 22

NUM_CORES = 2
NUM_SUBCORES = 16
NW = NUM_CORES * NUM_SUBCORES
LB = 128
CR = 8

NWORDS = NBITS // 32

GROUPS = 8
REPL = 4
GBITS = NBITS // GROUPS
GWORDS = GBITS // 32
SENT_LO = GWORDS * 32
VCH = 8192


def _scatter_body(vals_hbm, out_hbm, vbuf0, vbuf1, stage, bset, sem0, sem1):
  wid = lax.axis_index("s") * NUM_CORES + lax.axis_index("c")
  g = wid >> 2
  r = wid & 3
  lo0 = g * GBITS
  iota = jnp.arange(16, dtype=jnp.int32)
  DUMP = jnp.int32(GWORDS)
  SCAP = VCH // 16
  SDUMP = jnp.int32(16 * SCAP)

  @pl.loop(0, (GWORDS + 16) // 16)
  def _zero(z):
    bset[pl.ds(z * 16, 16)] = jnp.zeros((16,), jnp.int32)

  vals_per_repl = N_ADD_PAD // REPL
  base0 = r * vals_per_repl
  nchunks = vals_per_repl // VCH

  def _process(vbuf_s, sem_s, ci):
    pltpu.make_async_copy(vals_hbm.at[pl.ds(0, VCH)], vbuf_s, sem_s).wait()

    @pl.loop(0, VCH // 16, init_carry=jnp.zeros((16,), jnp.int32), unroll=4)
    def _scan(vi, cnt):
      v = vbuf_s[pl.ds(vi * 16, 16)]
      h = (v * jnp.int32(PRIME_I32)) & jnp.int32(MASK)
      lo = h - lo0
      inr = (lo >= 0) & (lo < GBITS)
      tgt = jnp.where(inr, cnt * 16 + iota, SDUMP + iota)
      plsc.store_scatter(stage, [tgt], lo)
      return cnt + jnp.where(inr, 1, 0)

    cnt = _scan
    nmax = cnt[0]
    for i in range(1, 16):
      nmax = jnp.maximum(nmax, cnt[i])

    @pl.loop(0, nmax)
    def _drain(e):
      raw = stage[pl.ds(e * 16, 16)]
      valid = e < cnt
      w = jnp.where(valid, raw >> 5, DUMP)
      m = jnp.where(valid, jnp.left_shift(jnp.int32(1), raw & 31), 0)
      w, m = lax.sort([w, m], num_keys=1)
      for s in (1, 2, 4, 8):
        pd = jnp.minimum(iota + s, 15)
        w_d = jnp.take_along_axis(w, pd, axis=0, mode="promise_in_bounds")
        m_d = jnp.take_along_axis(m, pd, axis=0, mode="promise_in_bounds")
        m = m | jnp.where(w_d == w, m_d, 0)
      w_u = jnp.take_along_axis(
          w, jnp.maximum(iota - 1, 0), axis=0, mode="promise_in_bounds")
      w = jnp.where((iota == 0) | (w != w_u), w, DUMP)
      cur = plsc.load_gather(bset, [w])
      plsc.store_scatter(bset, [w], cur | m)

    @pl.when(ci + 2 < nchunks)
    def _prefetch():
      pltpu.async_copy(
          vals_hbm.at[pl.ds(base0 + (ci + 2) * VCH, VCH)], vbuf_s, sem_s)

  vb0, vb1 = vbuf0, vbuf1
  pltpu.async_copy(vals_hbm.at[pl.ds(base0, VCH)], vb0, sem0)
  pltpu.async_copy(vals_hbm.at[pl.ds(base0 + VCH, VCH)], vb1, sem1)

  @pl.loop(0, nchunks // 2)
  def _chunk(pi):
    _process(vb0, sem0, pi * 2)
    _process(vb1, sem1, pi * 2 + 1)

  pltpu.sync_copy(bset.at[pl.ds(0, GWORDS)], out_hbm.at[wid])


def _gather_body(qvals_hbm, wp_hbm, out_hbm, qv, qidx, qbit, res, wsh, sem):
  cid = lax.axis_index("c")
  sid = lax.axis_index("s")
  wid = sid * NUM_CORES + cid
  rows_per_tile = N_Q // LB // NW
  row0 = wid * rows_per_tile

  stage = NWORDS // NUM_SUBCORES
  pltpu.sync_copy(wp_hbm.at[pl.ds(sid * stage, stage)],
                  wsh.at[pl.ds(sid * stage, stage)])
  plsc.subcore_barrier()

  @pl.loop(0, rows_per_tile // CR)
  def _chunk(ci):
    r = row0 + ci * CR
    pltpu.sync_copy(qvals_hbm.at[pl.ds(r, CR)], qv)
    for j in range(CR):
      for l in range(LB // 16):
        v = qv[j, pl.ds(l * 16, 16)]
        h = (v * jnp.int32(PRIME_I32)) & jnp.int32(MASK)
        qidx[j, pl.ds(l * 16, 16)] = h >> 5
        qbit[j, pl.ds(l * 16, 16)] = h & 31
    copies = [
        pltpu.async_copy(wsh.at[qidx.at[j]], res.at[j], sem)
        for j in range(CR)
    ]
    for cp in copies:
      cp.wait()
    for j in range(CR):
      for l in range(LB // 16):
        w = res[j, pl.ds(l * 16, 16)]
        b = qbit[j, pl.ds(l * 16, 16)]
        res[j, pl.ds(l * 16, 16)] = (w >> b) & 1
    pltpu.sync_copy(res, out_hbm.at[pl.ds(r, CR)])


@functools.cache
def _sc_kernels():
  mesh = plsc.VectorSubcoreMesh(
      core_axis_name="c", subcore_axis_name="s",
      num_cores=NUM_CORES, num_subcores=NUM_SUBCORES)
  scatter = pl.kernel(
      _scatter_body,
      out_type=jax.ShapeDtypeStruct((NW, GWORDS), jnp.int32),
      mesh=mesh,
      compiler_params=pltpu.CompilerParams(needs_layout_passes=False),
      scratch_types=[
          pltpu.VMEM((VCH,), jnp.int32),
          pltpu.VMEM((VCH,), jnp.int32),
          pltpu.VMEM((VCH + 16,), jnp.int32),
          pltpu.VMEM((GWORDS + 16,), jnp.int32),
          pltpu.SemaphoreType.DMA,
          pltpu.SemaphoreType.DMA,
      ],
  )
  gather = pl.kernel(
      _gather_body,
      out_type=jax.ShapeDtypeStruct((N_Q // LB, LB), jnp.int32),
      mesh=mesh,
      scratch_types=[
          pltpu.VMEM((CR, LB), jnp.int32),
          pltpu.VMEM((CR, LB), jnp.int32),
          pltpu.VMEM((CR, LB), jnp.int32),
          pltpu.VMEM((CR, LB), jnp.int32),
          pltpu.VMEM_SHARED((NWORDS,), jnp.int32),
          pltpu.SemaphoreType.DMA,
      ],
  )
  return scatter, gather


R2D = 512
C2D = 1024


def _window_body(x0, x1, x2, x3, o_ref):
  S = x0[...] | x1[...] | x2[...] | x3[...]
  cl = S[:, C2D - 1:]
  Sd = jnp.concatenate([cl[R2D - 1:], cl[:R2D - 1]], axis=0)
  P = jnp.concatenate([Sd, S[:, :C2D - 1]], axis=1)
  A = S
  for e in range(1, 7):
    A = A | (S << e) | lax.shift_right_logical(P, 32 - e)
  c0 = A[:, :1]
  Au = jnp.concatenate([c0[1:], c0[:1]], axis=0)
  Nx = jnp.concatenate([A[:, 1:], Au], axis=1)
  W = A
  for d in range(1, 7):
    W = W & (lax.shift_right_logical(A, d) | (Nx << (32 - d)))
  o_ref[...] = W


_window = pl.pallas_call(
    _window_body,
    out_shape=jax.ShapeDtypeStruct((R2D, C2D), jnp.int32),
)


def kernel(values_add, values_query):
  pad = jnp.broadcast_to(values_add[0], (N_ADD_PAD - N_ADD,))
  vals1d = jnp.concatenate([values_add, pad])
  q2d = values_query.reshape(N_Q // LB, LB)

  scatter_markers, gather_queries = _sc_kernels()
  reps32 = scatter_markers(vals1d)
  reps = reps32.reshape(GROUPS, REPL, GWORDS)
  rep = [reps[:, i, :].reshape(R2D, C2D) for i in range(REPL)]
  Wp = _window(rep[0], rep[1], rep[2], rep[3])

  out2d = gather_queries(q2d, Wp.reshape(-1))
  return out2d.reshape(-1) != 0


# speedup vs baseline: 20.7958x; 1.2826x over previous
import functools

import jax
import jax.numpy as jnp
from jax import lax
from jax.experimental import pallas as pl
from jax.experimental.pallas import tpu as pltpu
from jax.experimental.pallas import tpu_sc as plsc

NBITS = 1 << 24
MASK = NBITS - 1
PRIME_I32 = 2654435761 - (1 << 32)

N_ADD = 1_000_000
N_ADD_PAD = 1 << 20
N_Q = 1 << 22

NUM_CORES = 2
NUM_SUBCORES = 16
NW = NUM_CORES * NUM_SUBCORES
LB = 128
CR = 16

NWORDS = NBITS // 32

GROUPS = 8
REPL = 4
GBITS = NBITS // GROUPS
GWORDS = GBITS // 32
SENT_LO = GWORDS * 32
VCH = 8192


def _scatter_body(vals_hbm, out_hbm, vbuf0, vbuf1, stage, bset, sem0, sem1):
  wid = lax.axis_index("s") * NUM_CORES + lax.axis_index("c")
  g = wid >> 2
  r = wid & 3
  lo0 = g * GBITS
  iota = jnp.arange(16, dtype=jnp.int32)
  DUMP = jnp.int32(GWORDS)
  SCAP = VCH // 16
  SDUMP = jnp.int32(16 * SCAP)

  @pl.loop(0, (GWORDS + 16) // 16)
  def _zero(z):
    bset[pl.ds(z * 16, 16)] = jnp.zeros((16,), jnp.int32)

  vals_per_repl = N_ADD_PAD // REPL
  base0 = r * vals_per_repl
  nchunks = vals_per_repl // VCH

  def _process(vbuf_s, sem_s, ci):
    pltpu.make_async_copy(vals_hbm.at[pl.ds(0, VCH)], vbuf_s, sem_s).wait()

    @pl.loop(0, VCH // 16, init_carry=jnp.zeros((16,), jnp.int32), unroll=4)
    def _scan(vi, cnt):
      v = vbuf_s[pl.ds(vi * 16, 16)]
      h = (v * jnp.int32(PRIME_I32)) & jnp.int32(MASK)
      lo = h - lo0
      inr = (lo >= 0) & (lo < GBITS)
      tgt = jnp.where(inr, cnt * 16 + iota, SDUMP + iota)
      plsc.store_scatter(stage, [tgt], lo)
      return cnt + jnp.where(inr, 1, 0)

    cnt = _scan
    nmax = cnt[0]
    for i in range(1, 16):
      nmax = jnp.maximum(nmax, cnt[i])

    @pl.loop(0, nmax)
    def _drain(e):
      raw = stage[pl.ds(e * 16, 16)]
      valid = e < cnt
      w = jnp.where(valid, raw >> 5, DUMP)
      m = jnp.where(valid, jnp.left_shift(jnp.int32(1), raw & 31), 0)
      w, m = lax.sort([w, m], num_keys=1)
      for s in (1, 2, 4, 8):
        pd = jnp.minimum(iota + s, 15)
        w_d = jnp.take_along_axis(w, pd, axis=0, mode="promise_in_bounds")
        m_d = jnp.take_along_axis(m, pd, axis=0, mode="promise_in_bounds")
        m = m | jnp.where(w_d == w, m_d, 0)
      w_u = jnp.take_along_axis(
          w, jnp.maximum(iota - 1, 0), axis=0, mode="promise_in_bounds")
      w = jnp.where((iota == 0) | (w != w_u), w, DUMP)
      cur = plsc.load_gather(bset, [w])
      plsc.store_scatter(bset, [w], cur | m)

    @pl.when(ci + 2 < nchunks)
    def _prefetch():
      pltpu.async_copy(
          vals_hbm.at[pl.ds(base0 + (ci + 2) * VCH, VCH)], vbuf_s, sem_s)

  vb0, vb1 = vbuf0, vbuf1
  pltpu.async_copy(vals_hbm.at[pl.ds(base0, VCH)], vb0, sem0)
  pltpu.async_copy(vals_hbm.at[pl.ds(base0 + VCH, VCH)], vb1, sem1)

  @pl.loop(0, nchunks // 2)
  def _chunk(pi):
    _process(vb0, sem0, pi * 2)
    _process(vb1, sem1, pi * 2 + 1)

  pltpu.sync_copy(bset.at[pl.ds(0, GWORDS)],
                  out_hbm.at[r, pl.ds(g * GWORDS, GWORDS)])


def _gather_body(qvals_hbm, wp_hbm, out_hbm, qv0, qv1, qidx0, qidx1, qbit0,
                 qbit1, res0, res1, wsh, semv0, semv1, semg0, semg1, semo):
  cid = lax.axis_index("c")
  sid = lax.axis_index("s")
  wid = sid * NUM_CORES + cid
  rows_per_tile = N_Q // LB // NW
  row0 = wid * rows_per_tile
  nchunks = rows_per_tile // CR

  stage = NWORDS // NUM_SUBCORES
  pltpu.sync_copy(wp_hbm.at[pl.ds(sid * stage, stage)],
                  wsh.at[pl.ds(sid * stage, stage)])
  plsc.subcore_barrier()

  def _phase_a(ci, qv, qidx, qbit, res, semv, semg):
    pltpu.make_async_copy(qvals_hbm.at[pl.ds(0, CR)], qv, semv).wait()
    @pl.when(ci >= 2)
    def _drain_prev_out():
      pltpu.make_async_copy(res, out_hbm.at[pl.ds(0, CR)], semo).wait()
    for j in range(CR):
      for l in range(LB // 16):
        v = qv[j, pl.ds(l * 16, 16)]
        h = (v * jnp.int32(PRIME_I32)) & jnp.int32(MASK)
        qidx[j, pl.ds(l * 16, 16)] = h >> 5
        qbit[j, pl.ds(l * 16, 16)] = h & 31
    for j in range(CR):
      pltpu.async_copy(wsh.at[qidx.at[j]], res.at[j], semg)

    @pl.when(ci + 2 < nchunks)
    def _prefetch():
      pltpu.async_copy(
          qvals_hbm.at[pl.ds(row0 + (ci + 2) * CR, CR)], qv, semv)

  def _phase_b(ci, qidx, qbit, res, semg):
    for j in range(CR):
      pltpu.make_async_copy(wsh.at[qidx.at[j]], res.at[j], semg).wait()
    for j in range(CR):
      for l in range(LB // 16):
        w = res[j, pl.ds(l * 16, 16)]
        b = qbit[j, pl.ds(l * 16, 16)]
        res[j, pl.ds(l * 16, 16)] = (w >> b) & 1
    pltpu.async_copy(res, out_hbm.at[pl.ds(row0 + ci * CR, CR)], semo)

  pltpu.async_copy(qvals_hbm.at[pl.ds(row0, CR)], qv0, semv0)
  pltpu.async_copy(qvals_hbm.at[pl.ds(row0 + CR, CR)], qv1, semv1)
  _phase_a(0, qv0, qidx0, qbit0, res0, semv0, semg0)

  @pl.loop(0, nchunks // 2)
  def _chunk(pi):
    c0 = pi * 2
    c1 = c0 + 1
    _phase_a(c1, qv1, qidx1, qbit1, res1, semv1, semg1)
    _phase_b(c0, qidx0, qbit0, res0, semg0)

    @pl.when(c0 + 2 < nchunks)
    def _next_a():
      _phase_a(c0 + 2, qv0, qidx0, qbit0, res0, semv0, semg0)

    _phase_b(c1, qidx1, qbit1, res1, semg1)

  pltpu.make_async_copy(res0, out_hbm.at[pl.ds(0, CR)], semo).wait()
  pltpu.make_async_copy(res1, out_hbm.at[pl.ds(0, CR)], semo).wait()


@functools.cache
def _sc_kernels():
  mesh = plsc.VectorSubcoreMesh(
      core_axis_name="c", subcore_axis_name="s",
      num_cores=NUM_CORES, num_subcores=NUM_SUBCORES)
  scatter = pl.kernel(
      _scatter_body,
      out_type=jax.ShapeDtypeStruct((REPL, NWORDS), jnp.int32),
      mesh=mesh,
      compiler_params=pltpu.CompilerParams(needs_layout_passes=False),
      scratch_types=[
          pltpu.VMEM((VCH,), jnp.int32),
          pltpu.VMEM((VCH,), jnp.int32),
          pltpu.VMEM((VCH + 16,), jnp.int32),
          pltpu.VMEM((GWORDS + 16,), jnp.int32),
          pltpu.SemaphoreType.DMA,
          pltpu.SemaphoreType.DMA,
      ],
  )
  gather = pl.kernel(
      _gather_body,
      out_type=jax.ShapeDtypeStruct((N_Q // LB, LB), jnp.int32),
      mesh=mesh,
      scratch_types=[
          pltpu.VMEM((CR, LB), jnp.int32),
          pltpu.VMEM((CR, LB), jnp.int32),
          pltpu.VMEM((CR, LB), jnp.int32),
          pltpu.VMEM((CR, LB), jnp.int32),
          pltpu.VMEM((CR, LB), jnp.int32),
          pltpu.VMEM((CR, LB), jnp.int32),
          pltpu.VMEM((CR, LB), jnp.int32),
          pltpu.VMEM((CR, LB), jnp.int32),
          pltpu.VMEM_SHARED((NWORDS,), jnp.int32),
          pltpu.SemaphoreType.DMA,
          pltpu.SemaphoreType.DMA,
          pltpu.SemaphoreType.DMA,
          pltpu.SemaphoreType.DMA,
          pltpu.SemaphoreType.DMA,
      ],
  )
  return scatter, gather


R2D = 512
C2D = 1024


def _window_body(x0, x1, x2, x3, o_ref):
  S = x0[...] | x1[...] | x2[...] | x3[...]
  cl = S[:, C2D - 1:]
  Sd = jnp.concatenate([cl[R2D - 1:], cl[:R2D - 1]], axis=0)
  P = jnp.concatenate([Sd, S[:, :C2D - 1]], axis=1)
  A = S
  for e in range(1, 7):
    A = A | (S << e) | lax.shift_right_logical(P, 32 - e)
  c0 = A[:, :1]
  Au = jnp.concatenate([c0[1:], c0[:1]], axis=0)
  Nx = jnp.concatenate([A[:, 1:], Au], axis=1)
  W = A
  for d in range(1, 7):
    W = W & (lax.shift_right_logical(A, d) | (Nx << (32 - d)))
  o_ref[...] = W


_window = pl.pallas_call(
    _window_body,
    out_shape=jax.ShapeDtypeStruct((R2D, C2D), jnp.int32),
)


def kernel(values_add, values_query):
  pad = jnp.broadcast_to(values_add[0], (N_ADD_PAD - N_ADD,))
  vals1d = jnp.concatenate([values_add, pad])
  q2d = values_query.reshape(N_Q // LB, LB)

  scatter_markers, gather_queries = _sc_kernels()
  reps32 = scatter_markers(vals1d)
  rep = [reps32[i].reshape(R2D, C2D) for i in range(REPL)]
  Wp = _window(rep[0], rep[1], rep[2], rep[3])

  out2d = gather_queries(q2d, Wp.reshape(-1))
  return out2d.reshape(-1) != 0


# speedup vs baseline: 20.8167x; 1.0010x over previous
import functools

import jax
import jax.numpy as jnp
from jax import lax
from jax.experimental import pallas as pl
from jax.experimental.pallas import tpu as pltpu
from jax.experimental.pallas import tpu_sc as plsc

NBITS = 1 << 24
MASK = NBITS - 1
PRIME_I32 = 2654435761 - (1 << 32)

N_ADD = 1_000_000
N_ADD_PAD = 1 << 20
N_Q = 1 << 22

NUM_CORES = 2
NUM_SUBCORES = 16
NW = NUM_CORES * NUM_SUBCORES
LB = 128
CR = 16

NWORDS = NBITS // 32

GROUPS = 8
REPL = 4
GBITS = NBITS // GROUPS
GWORDS = GBITS // 32
SENT_LO = GWORDS * 32
VCH = 8192


def _scatter_body(vals_hbm, out_hbm, vbuf0, vbuf1, stage, bset, sem0, sem1):
  wid = lax.axis_index("s") * NUM_CORES + lax.axis_index("c")
  g = wid >> 2
  r = wid & 3
  lo0 = g * GBITS
  iota = jnp.arange(16, dtype=jnp.int32)
  DUMP = jnp.int32(GWORDS)
  SCAP = VCH // 16
  SDUMP = jnp.int32(16 * SCAP)

  @pl.loop(0, (GWORDS + 16) // 16)
  def _zero(z):
    bset[pl.ds(z * 16, 16)] = jnp.zeros((16,), jnp.int32)

  vals_per_repl = N_ADD_PAD // REPL
  base0 = r * vals_per_repl
  nchunks = vals_per_repl // VCH

  def _process(vbuf_s, sem_s, ci):
    pltpu.make_async_copy(vals_hbm.at[pl.ds(0, VCH)], vbuf_s, sem_s).wait()

    @pl.loop(0, VCH // 16, init_carry=jnp.zeros((16,), jnp.int32), unroll=8)
    def _scan(vi, cnt):
      v = vbuf_s[pl.ds(vi * 16, 16)]
      h = (v * jnp.int32(PRIME_I32)) & jnp.int32(MASK)
      lo = h - lo0
      inr = plsc.bitcast(lo, jnp.uint32) < jnp.uint32(GBITS)
      tgt = jnp.where(inr, cnt * 16 + iota, SDUMP + iota)
      plsc.store_scatter(stage, [tgt], lo)
      return cnt + jnp.where(inr, 1, 0)

    cnt = _scan
    nmax = cnt[0]
    for i in range(1, 16):
      nmax = jnp.maximum(nmax, cnt[i])

    def _rmw_one(e):
      raw = stage[pl.ds(e * 16, 16)]
      valid = e < cnt
      w = jnp.where(valid, raw >> 5, DUMP)
      m = jnp.where(valid, jnp.left_shift(jnp.int32(1), raw & 31), 0)
      w, m = lax.sort([w, m], num_keys=1)
      for s in (1, 2, 4, 8):
        pd = jnp.minimum(iota + s, 15)
        w_d = jnp.take_along_axis(w, pd, axis=0, mode="promise_in_bounds")
        m_d = jnp.take_along_axis(m, pd, axis=0, mode="promise_in_bounds")
        m = m | jnp.where(w_d == w, m_d, 0)
      w_u = jnp.take_along_axis(
          w, jnp.maximum(iota - 1, 0), axis=0, mode="promise_in_bounds")
      w = jnp.where((iota == 0) | (w != w_u), w, DUMP)
      cur = plsc.load_gather(bset, [w])
      plsc.store_scatter(bset, [w], cur | m)

    @pl.loop(0, (nmax + 1) >> 1)
    def _drain(p):
      _rmw_one(p * 2)
      _rmw_one(p * 2 + 1)

    @pl.when(ci + 2 < nchunks)
    def _prefetch():
      pltpu.async_copy(
          vals_hbm.at[pl.ds(base0 + (ci + 2) * VCH, VCH)], vbuf_s, sem_s)

  vb0, vb1 = vbuf0, vbuf1
  pltpu.async_copy(vals_hbm.at[pl.ds(base0, VCH)], vb0, sem0)
  pltpu.async_copy(vals_hbm.at[pl.ds(base0 + VCH, VCH)], vb1, sem1)

  @pl.loop(0, nchunks // 2)
  def _chunk(pi):
    _process(vb0, sem0, pi * 2)
    _process(vb1, sem1, pi * 2 + 1)

  pltpu.sync_copy(bset.at[pl.ds(0, GWORDS)],
                  out_hbm.at[r, pl.ds(g * GWORDS, GWORDS)])


def _gather_body(qvals_hbm, wp_hbm, out_hbm, qv0, qv1, qidx0, qidx1, qbit0,
                 qbit1, res0, res1, wsh, semv0, semv1, semg0, semg1, semo):
  cid = lax.axis_index("c")
  sid = lax.axis_index("s")
  wid = sid * NUM_CORES + cid
  rows_per_tile = N_Q // LB // NW
  row0 = wid * rows_per_tile
  nchunks = rows_per_tile // CR

  stage = NWORDS // NUM_SUBCORES
  pltpu.sync_copy(wp_hbm.at[pl.ds(sid * stage, stage)],
                  wsh.at[pl.ds(sid * stage, stage)])
  plsc.subcore_barrier()

  def _phase_a(ci, qv, qidx, qbit, res, semv, semg):
    pltpu.make_async_copy(qvals_hbm.at[pl.ds(0, CR)], qv, semv).wait()
    @pl.when(ci >= 2)
    def _drain_prev_out():
      pltpu.make_async_copy(res, out_hbm.at[pl.ds(0, CR)], semo).wait()
    for j in range(CR):
      for l in range(LB // 16):
        v = qv[j, pl.ds(l * 16, 16)]
        h = (v * jnp.int32(PRIME_I32)) & jnp.int32(MASK)
        qidx[j, pl.ds(l * 16, 16)] = h >> 5
        qbit[j, pl.ds(l * 16, 16)] = h & 31
    for j in range(CR):
      pltpu.async_copy(wsh.at[qidx.at[j]], res.at[j], semg)

    @pl.when(ci + 2 < nchunks)
    def _prefetch():
      pltpu.async_copy(
          qvals_hbm.at[pl.ds(row0 + (ci + 2) * CR, CR)], qv, semv)

  def _phase_b(ci, qidx, qbit, res, semg):
    for j in range(CR):
      pltpu.make_async_copy(wsh.at[qidx.at[j]], res.at[j], semg).wait()
    for j in range(CR):
      for l in range(LB // 16):
        w = res[j, pl.ds(l * 16, 16)]
        b = qbit[j, pl.ds(l * 16, 16)]
        res[j, pl.ds(l * 16, 16)] = (w >> b) & 1
    pltpu.async_copy(res, out_hbm.at[pl.ds(row0 + ci * CR, CR)], semo)

  pltpu.async_copy(qvals_hbm.at[pl.ds(row0, CR)], qv0, semv0)
  pltpu.async_copy(qvals_hbm.at[pl.ds(row0 + CR, CR)], qv1, semv1)
  _phase_a(0, qv0, qidx0, qbit0, res0, semv0, semg0)

  @pl.loop(0, nchunks // 2)
  def _chunk(pi):
    c0 = pi * 2
    c1 = c0 + 1
    _phase_a(c1, qv1, qidx1, qbit1, res1, semv1, semg1)
    _phase_b(c0, qidx0, qbit0, res0, semg0)

    @pl.when(c0 + 2 < nchunks)
    def _next_a():
      _phase_a(c0 + 2, qv0, qidx0, qbit0, res0, semv0, semg0)

    _phase_b(c1, qidx1, qbit1, res1, semg1)

  pltpu.make_async_copy(res0, out_hbm.at[pl.ds(0, CR)], semo).wait()
  pltpu.make_async_copy(res1, out_hbm.at[pl.ds(0, CR)], semo).wait()


@functools.cache
def _sc_kernels():
  mesh = plsc.VectorSubcoreMesh(
      core_axis_name="c", subcore_axis_name="s",
      num_cores=NUM_CORES, num_subcores=NUM_SUBCORES)
  scatter = pl.kernel(
      _scatter_body,
      out_type=jax.ShapeDtypeStruct((REPL, NWORDS), jnp.int32),
      mesh=mesh,
      compiler_params=pltpu.CompilerParams(needs_layout_passes=False),
      scratch_types=[
          pltpu.VMEM((VCH,), jnp.int32),
          pltpu.VMEM((VCH,), jnp.int32),
          pltpu.VMEM((VCH + 16,), jnp.int32),
          pltpu.VMEM((GWORDS + 16,), jnp.int32),
          pltpu.SemaphoreType.DMA,
          pltpu.SemaphoreType.DMA,
      ],
  )
  gather = pl.kernel(
      _gather_body,
      out_type=jax.ShapeDtypeStruct((N_Q // LB, LB), jnp.int32),
      mesh=mesh,
      scratch_types=[
          pltpu.VMEM((CR, LB), jnp.int32),
          pltpu.VMEM((CR, LB), jnp.int32),
          pltpu.VMEM((CR, LB), jnp.int32),
          pltpu.VMEM((CR, LB), jnp.int32),
          pltpu.VMEM((CR, LB), jnp.int32),
          pltpu.VMEM((CR, LB), jnp.int32),
          pltpu.VMEM((CR, LB), jnp.int32),
          pltpu.VMEM((CR, LB), jnp.int32),
          pltpu.VMEM_SHARED((NWORDS,), jnp.int32),
          pltpu.SemaphoreType.DMA,
          pltpu.SemaphoreType.DMA,
          pltpu.SemaphoreType.DMA,
          pltpu.SemaphoreType.DMA,
          pltpu.SemaphoreType.DMA,
      ],
  )
  return scatter, gather


R2D = 512
C2D = 1024


def _window_body(x0, x1, x2, x3, o_ref):
  S = x0[...] | x1[...] | x2[...] | x3[...]
  cl = S[:, C2D - 1:]
  Sd = jnp.concatenate([cl[R2D - 1:], cl[:R2D - 1]], axis=0)
  P = jnp.concatenate([Sd, S[:, :C2D - 1]], axis=1)
  A = S
  for e in range(1, 7):
    A = A | (S << e) | lax.shift_right_logical(P, 32 - e)
  c0 = A[:, :1]
  Au = jnp.concatenate([c0[1:], c0[:1]], axis=0)
  Nx = jnp.concatenate([A[:, 1:], Au], axis=1)
  W = A
  for d in range(1, 7):
    W = W & (lax.shift_right_logical(A, d) | (Nx << (32 - d)))
  o_ref[...] = W


_window = pl.pallas_call(
    _window_body,
    out_shape=jax.ShapeDtypeStruct((R2D, C2D), jnp.int32),
)


def kernel(values_add, values_query):
  pad = jnp.broadcast_to(values_add[0], (N_ADD_PAD - N_ADD,))
  vals1d = jnp.concatenate([values_add, pad])
  q2d = values_query.reshape(N_Q // LB, LB)

  scatter_markers, gather_queries = _sc_kernels()
  reps32 = scatter_markers(vals1d)
  rep = [reps32[i].reshape(R2D, C2D) for i in range(REPL)]
  Wp = _window(rep[0], rep[1], rep[2], rep[3])

  out2d = gather_queries(q2d, Wp.reshape(-1))
  return out2d.reshape(-1) != 0


# speedup vs baseline: 20.8269x; 1.0005x over previous
import functools

import jax
import jax.numpy as jnp
from jax import lax
from jax.experimental import pallas as pl
from jax.experimental.pallas import tpu as pltpu
from jax.experimental.pallas import tpu_sc as plsc

NBITS = 1 << 24
MASK = NBITS - 1
PRIME_I32 = 2654435761 - (1 << 32)

N_ADD = 1_000_000
N_ADD_PAD = 1 << 20
N_Q = 1 << 22

NUM_CORES = 2
NUM_SUBCORES = 16
NW = NUM_CORES * NUM_SUBCORES
LB = 128
CR = 16

NWORDS = NBITS // 32

GROUPS = 8
REPL = 4
GBITS = NBITS // GROUPS
GWORDS = GBITS // 32
VCH = 8192


def _scatter_body(vals_hbm, out_hbm, vbuf0, vbuf1, stage, bset, sem0, sem1):
  wid = lax.axis_index("s") * NUM_CORES + lax.axis_index("c")
  g = wid >> 2
  r = wid & 3
  lo0 = g * GBITS
  iota = jnp.arange(16, dtype=jnp.int32)
  DUMP = jnp.int32(GWORDS)
  SCAP = VCH // 16
  SDUMP = jnp.int32(16 * SCAP)

  @pl.loop(0, (GWORDS + 16) // 16)
  def _zero(z):
    bset[pl.ds(z * 16, 16)] = jnp.zeros((16,), jnp.int32)

  vals_per_repl = N_ADD_PAD // REPL
  base0 = r * vals_per_repl
  nchunks = vals_per_repl // VCH

  def _process(vbuf_s, sem_s, ci):
    pltpu.make_async_copy(vals_hbm.at[pl.ds(0, VCH)], vbuf_s, sem_s).wait()

    @pl.loop(0, VCH // 16, init_carry=jnp.zeros((16,), jnp.int32), unroll=8)
    def _scan(vi, cnt):
      v = vbuf_s[pl.ds(vi * 16, 16)]
      h = (v * jnp.int32(PRIME_I32)) & jnp.int32(MASK)
      lo = h - lo0
      inr = plsc.bitcast(lo, jnp.uint32) < jnp.uint32(GBITS)
      tgt = jnp.where(inr, cnt * 16 + iota, SDUMP + iota)
      plsc.store_scatter(stage, [tgt], lo)
      return cnt + jnp.where(inr, 1, 0)

    cnt = _scan
    nmax = cnt[0]
    for i in range(1, 16):
      nmax = jnp.maximum(nmax, cnt[i])

    def _rmw_one(e):
      raw = stage[pl.ds(e * 16, 16)]
      valid = e < cnt
      w = jnp.where(valid, raw >> 5, DUMP)
      m = jnp.where(valid, jnp.left_shift(jnp.int32(1), raw & 31), 0)
      w, m = lax.sort([w, m], num_keys=1)
      for s in (1, 2, 4, 8):
        pd = jnp.minimum(iota + s, 15)
        w_d = jnp.take_along_axis(w, pd, axis=0, mode="promise_in_bounds")
        m_d = jnp.take_along_axis(m, pd, axis=0, mode="promise_in_bounds")
        m = m | jnp.where(w_d == w, m_d, 0)
      w_u = jnp.take_along_axis(
          w, jnp.maximum(iota - 1, 0), axis=0, mode="promise_in_bounds")
      w = jnp.where((iota == 0) | (w != w_u), w, DUMP)
      cur = plsc.load_gather(bset, [w])
      plsc.store_scatter(bset, [w], cur | m)

    @pl.loop(0, (nmax + 1) >> 1)
    def _drain(p):
      _rmw_one(p * 2)
      _rmw_one(p * 2 + 1)

    @pl.when(ci + 2 < nchunks)
    def _prefetch():
      pltpu.async_copy(
          vals_hbm.at[pl.ds(base0 + (ci + 2) * VCH, VCH)], vbuf_s, sem_s)

  vb0, vb1 = vbuf0, vbuf1
  pltpu.async_copy(vals_hbm.at[pl.ds(base0, VCH)], vb0, sem0)
  pltpu.async_copy(vals_hbm.at[pl.ds(base0 + VCH, VCH)], vb1, sem1)

  @pl.loop(0, nchunks // 2)
  def _chunk(pi):
    _process(vb0, sem0, pi * 2)
    _process(vb1, sem1, pi * 2 + 1)

  pltpu.sync_copy(bset.at[pl.ds(0, GWORDS)],
                  out_hbm.at[r, pl.ds(g * GWORDS, GWORDS)])


def _gather_body(qvals_hbm, wp_hbm, out_hbm, qv0, qv1, qidx0, qidx1, qbit0,
                 qbit1, res0, res1, wsh, semv0, semv1, semg0, semg1, semo):
  cid = lax.axis_index("c")
  sid = lax.axis_index("s")
  wid = sid * NUM_CORES + cid
  rows_per_tile = N_Q // LB // NW
  row0 = wid * rows_per_tile
  nchunks = rows_per_tile // CR

  stage = NWORDS // NUM_SUBCORES
  pltpu.sync_copy(wp_hbm.at[pl.ds(sid * stage, stage)],
                  wsh.at[pl.ds(sid * stage, stage)])
  plsc.subcore_barrier()

  def _phase_a(ci, qv, qidx, qbit, res, semv, semg):
    pltpu.make_async_copy(qvals_hbm.at[pl.ds(0, CR)], qv, semv).wait()
    @pl.when(ci >= 2)
    def _drain_prev_out():
      pltpu.make_async_copy(res, out_hbm.at[pl.ds(0, CR)], semo).wait()
    for j in range(CR):
      for l in range(LB // 16):
        v = qv[j, pl.ds(l * 16, 16)]
        h = (v * jnp.int32(PRIME_I32)) & jnp.int32(MASK)
        qidx[j, pl.ds(l * 16, 16)] = h >> 5
        qbit[j, pl.ds(l * 16, 16)] = h & 31
    for j in range(CR):
      pltpu.async_copy(wsh.at[qidx.at[j]], res.at[j], semg)

    @pl.when(ci + 2 < nchunks)
    def _prefetch():
      pltpu.async_copy(
          qvals_hbm.at[pl.ds(row0 + (ci + 2) * CR, CR)], qv, semv)

  def _phase_b(ci, qidx, qbit, res, semg):
    for j in range(CR):
      pltpu.make_async_copy(wsh.at[qidx.at[j]], res.at[j], semg).wait()
    for j in range(CR):
      for l in range(LB // 16):
        w = res[j, pl.ds(l * 16, 16)]
        b = qbit[j, pl.ds(l * 16, 16)]
        res[j, pl.ds(l * 16, 16)] = (w >> b) & 1
    pltpu.async_copy(res, out_hbm.at[pl.ds(row0 + ci * CR, CR)], semo)

  pltpu.async_copy(qvals_hbm.at[pl.ds(row0, CR)], qv0, semv0)
  pltpu.async_copy(qvals_hbm.at[pl.ds(row0 + CR, CR)], qv1, semv1)
  _phase_a(0, qv0, qidx0, qbit0, res0, semv0, semg0)

  @pl.loop(0, nchunks // 2)
  def _chunk(pi):
    c0 = pi * 2
    c1 = c0 + 1
    _phase_a(c1, qv1, qidx1, qbit1, res1, semv1, semg1)
    _phase_b(c0, qidx0, qbit0, res0, semg0)

    @pl.when(c0 + 2 < nchunks)
    def _next_a():
      _phase_a(c0 + 2, qv0, qidx0, qbit0, res0, semv0, semg0)

    _phase_b(c1, qidx1, qbit1, res1, semg1)

  pltpu.make_async_copy(res0, out_hbm.at[pl.ds(0, CR)], semo).wait()
  pltpu.make_async_copy(res1, out_hbm.at[pl.ds(0, CR)], semo).wait()


@functools.cache
def _sc_kernels():
  mesh = plsc.VectorSubcoreMesh(
      core_axis_name="c", subcore_axis_name="s",
      num_cores=NUM_CORES, num_subcores=NUM_SUBCORES)
  scatter = pl.kernel(
      _scatter_body,
      out_type=jax.ShapeDtypeStruct((REPL, NWORDS), jnp.int32),
      mesh=mesh,
      compiler_params=pltpu.CompilerParams(needs_layout_passes=False),
      scratch_types=[
          pltpu.VMEM((VCH,), jnp.int32),
          pltpu.VMEM((VCH,), jnp.int32),
          pltpu.VMEM((VCH + 16,), jnp.int32),
          pltpu.VMEM((GWORDS + 16,), jnp.int32),
          pltpu.SemaphoreType.DMA,
          pltpu.SemaphoreType.DMA,
      ],
  )
  gather = pl.kernel(
      _gather_body,
      out_type=jax.ShapeDtypeStruct((N_Q // LB, LB), jnp.int32),
      mesh=mesh,
      scratch_types=[
          pltpu.VMEM((CR, LB), jnp.int32),
          pltpu.VMEM((CR, LB), jnp.int32),
          pltpu.VMEM((CR, LB), jnp.int32),
          pltpu.VMEM((CR, LB), jnp.int32),
          pltpu.VMEM((CR, LB), jnp.int32),
          pltpu.VMEM((CR, LB), jnp.int32),
          pltpu.VMEM((CR, LB), jnp.int32),
          pltpu.VMEM((CR, LB), jnp.int32),
          pltpu.VMEM_SHARED((NWORDS,), jnp.int32),
          pltpu.SemaphoreType.DMA,
          pltpu.SemaphoreType.DMA,
          pltpu.SemaphoreType.DMA,
          pltpu.SemaphoreType.DMA,
          pltpu.SemaphoreType.DMA,
      ],
  )
  return scatter, gather


R2D = 512
C2D = 1024


def _window_body(x0, x1, x2, x3, o_ref):
  S = x0[...] | x1[...] | x2[...] | x3[...]
  cl = S[:, C2D - 1:]
  Sd = jnp.concatenate([cl[R2D - 1:], cl[:R2D - 1]], axis=0)
  P = jnp.concatenate([Sd, S[:, :C2D - 1]], axis=1)
  A = S
  for e in range(1, 7):
    A = A | (S << e) | lax.shift_right_logical(P, 32 - e)
  c0 = A[:, :1]
  Au = jnp.concatenate([c0[1:], c0[:1]], axis=0)
  Nx = jnp.concatenate([A[:, 1:], Au], axis=1)
  W = A
  for d in range(1, 7):
    W = W & (lax.shift_right_logical(A, d) | (Nx << (32 - d)))
  o_ref[...] = W


_window = pl.pallas_call(
    _window_body,
    out_shape=jax.ShapeDtypeStruct((R2D, C2D), jnp.int32),
)


def kernel(values_add, values_query):
  pad = jnp.broadcast_to(values_add[0], (N_ADD_PAD - N_ADD,))
  vals1d = jnp.concatenate([values_add, pad])
  q2d = values_query.reshape(N_Q // LB, LB)

  scatter_markers, gather_queries = _sc_kernels()
  reps32 = scatter_markers(vals1d)
  rep = [reps32[i].reshape(R2D, C2D) for i in range(REPL)]
  Wp = _window(rep[0], rep[1], rep[2], rep[3])

  out2d = gather_queries(q2d, Wp.reshape(-1))
  return out2d.reshape(-1) != 0
